# baseline (speedup 1.0000x reference)
import functools

import jax
import jax.numpy as jnp
from jax import lax
from jax.experimental import pallas as pl
from jax.experimental.pallas import tpu as pltpu
from jax.experimental.pallas import tpu_sc as plsc

NUM_LODS = 10
TEX_H = 512
TEX_W = 512
NCH = 11
BATCH = 1048576

CPAD = 16
TSLAB = 1 << 19

NC = 2
NS = 16
L = 16
NW = NC * NS

BPW = BATCH // NW
CHUNK = 2048
NCHUNK = BPW // CHUNK

_mesh = plsc.VectorSubcoreMesh(core_axis_name="c", subcore_axis_name="s")


def _iota():
    return lax.iota(jnp.int32, L)


_GDN = lax.GatherDimensionNumbers(
    offset_dims=(), collapsed_slice_dims=(0,), start_index_map=(0,))


def _rot(v, sh):
    idx = (_iota() + (sh % L)) & (L - 1)
    return lax.gather(v, idx[:, None], _GDN, (1,),
                      mode=lax.GatherScatterMode.PROMISE_IN_BOUNDS)


@functools.partial(
    pl.kernel,
    out_type=jax.ShapeDtypeStruct((BATCH * NCH,), jnp.float32),
    mesh=_mesh,
    compiler_params=pltpu.CompilerParams(use_tc_tiling_on_sc=False),
    scratch_types=[
        pltpu.HBM((NC * TSLAB, CPAD), jnp.float32),
        [pltpu.VMEM((TEX_W * NCH,), jnp.float32) for _ in range(2)],
        pltpu.VMEM((TEX_W, CPAD), jnp.float32),
        [pltpu.VMEM((CHUNK * 3,), jnp.int32) for _ in range(2)],
        [pltpu.VMEM((CHUNK,), jnp.int32) for _ in range(2)],
        [pltpu.VMEM((CHUNK, CPAD), jnp.float32) for _ in range(2)],
        pltpu.VMEM((CHUNK * NCH,), jnp.float32),
        pltpu.SemaphoreType.DMA,
        [pltpu.SemaphoreType.DMA for _ in range(2)],
        [pltpu.SemaphoreType.DMA for _ in range(2)],
    ],
)
def _tex_gather(r0, r1, r2, r3, r4, r5, r6, rtail, ys, xs, lods, out,
                table, stage_in, stage_out, bi_v, idx_v, rows_v, pack_v,
                rep_sem, bi_sem, gat_sem):
    regions = (r0, r1, r2, r3, r4, r5, r6)
    cid = lax.axis_index("c")
    sid = lax.axis_index("s")
    wid = sid * NC + cid
    slab = cid * TSLAB

    def expand_group(g, ntex, sbuf, lane_off=0):
        a = [sbuf[pl.ds(lane_off + g * (L * NCH) + L * v, L)]
             for v in range(NCH)]
        for jj in range(min(L, ntex)):
            off = (NCH * jj) % L
            v0 = (NCH * jj) // L
            r = _rot(a[v0], off)
            if off > L - NCH:
                r = jnp.where(_iota() < (L - off), r, _rot(a[v0 + 1], off))
            stage_out[g * L + jj, :] = r

    def expand_and_store(y, wk, base_k, sbuf):
        if wk >= L:
            def grp(g, _):
                expand_group(g, L, sbuf)
                return 0
            lax.fori_loop(0, wk // L, grp, 0)
        else:
            expand_group(0, wk, sbuf)
        pltpu.sync_copy(
            stage_out.at[pl.ds(0, wk)],
            table.at[pl.ds(slab + base_k + y * wk, wk)],
        )

    for k in range(7):
        wk = TEX_W >> k
        base_k = TSLAB - (TSLAB >> k)
        reg = regions[k]
        n = wk * NCH
        if wk >= NS:
            rows_per = wk // NS
            y0 = sid * rows_per
            if rows_per >= 2:
                h = pltpu.async_copy(
                    reg.at[pl.ds(y0 * n, n)],
                    stage_in[0].at[pl.ds(0, n)], rep_sem)

                def rep2(r2i, _, wk=wk, base_k=base_k, reg=reg, n=n, y0=y0,
                         rows_per=rows_per):
                    y = y0 + r2i * 2
                    pltpu.make_async_copy(
                        reg.at[pl.ds(y * n, n)],
                        stage_in[0].at[pl.ds(0, n)], rep_sem).wait()
                    pltpu.async_copy(
                        reg.at[pl.ds((y + 1) * n, n)],
                        stage_in[1].at[pl.ds(0, n)], rep_sem)
                    expand_and_store(y, wk, base_k, stage_in[0])
                    pltpu.make_async_copy(
                        reg.at[pl.ds((y + 1) * n, n)],
                        stage_in[1].at[pl.ds(0, n)], rep_sem).wait()

                    @pl.when(r2i + 1 < rows_per // 2)
                    def _():
                        pltpu.async_copy(
                            reg.at[pl.ds((y + 2) * n, n)],
                            stage_in[0].at[pl.ds(0, n)], rep_sem)
                    expand_and_store(y + 1, wk, base_k, stage_in[1])
                    return 0

                lax.fori_loop(0, rows_per // 2, rep2, 0)
            else:
                pltpu.sync_copy(reg.at[pl.ds(y0 * n, n)],
                                stage_in[0].at[pl.ds(0, n)])
                expand_and_store(y0, wk, base_k, stage_in[0])
        else:
            @pl.when(sid < wk)
            def _(wk=wk, base_k=base_k, reg=reg, n=n):
                pltpu.sync_copy(reg.at[pl.ds(sid * n, n)],
                                stage_in[0].at[pl.ds(0, n)])
                expand_and_store(sid, wk, base_k, stage_in[0])

    @pl.when(sid == 0)
    def _():
        pltpu.sync_copy(rtail.at[pl.ds(0, 176)], stage_in[0].at[pl.ds(0, 176)])
        expand_group(0, 16, stage_in[0])
        pltpu.sync_copy(stage_out.at[pl.ds(0, 16)],
                        table.at[pl.ds(slab + TSLAB - (TSLAB >> 7), 16)])

    @pl.when(sid == 1)
    def _():
        pltpu.sync_copy(rtail.at[pl.ds(176, 48)], stage_in[0].at[pl.ds(0, 48)])
        expand_group(0, 4, stage_in[0])
        pltpu.sync_copy(stage_out.at[pl.ds(0, 4)],
                        table.at[pl.ds(slab + TSLAB - (TSLAB >> 8), 4)])

    @pl.when(sid == 2)
    def _():
        pltpu.sync_copy(rtail.at[pl.ds(216, 16)], stage_in[0].at[pl.ds(0, 16)])
        stage_out[0, :] = _rot(stage_in[0][pl.ds(0, L)], 4)
        pltpu.sync_copy(stage_out.at[pl.ds(0, 1)],
                        table.at[pl.ds(slab + TSLAB - (TSLAB >> 9), 1)])

    plsc.subcore_barrier()

    base = wid * BPW

    def start_bi(ci, b):
        cbase = base + ci * CHUNK
        return (
            pltpu.async_copy(ys.at[pl.ds(cbase, CHUNK)],
                             bi_v[b].at[pl.ds(0, CHUNK)], bi_sem[b]),
            pltpu.async_copy(xs.at[pl.ds(cbase, CHUNK)],
                             bi_v[b].at[pl.ds(CHUNK, CHUNK)], bi_sem[b]),
            pltpu.async_copy(lods.at[pl.ds(cbase, CHUNK)],
                             bi_v[b].at[pl.ds(2 * CHUNK, CHUNK)], bi_sem[b]),
        )

    def compute_idx(b):
        def vec_body(vi, _):
            s = pl.ds(vi * L, L)
            y = bi_v[b][s]
            x = bi_v[b][pl.ds(CHUNK + vi * L, L)]
            ld = bi_v[b][pl.ds(2 * CHUNK + vi * L, L)]
            lbase = TSLAB - lax.shift_right_logical(
                jnp.full((L,), TSLAB, jnp.int32), ld)
            w = lax.shift_right_logical(jnp.full((L,), TEX_W, jnp.int32), ld)
            idx = (
                slab + lbase
                + lax.shift_right_logical(y, ld) * w
                + lax.shift_right_logical(x, ld)
            )
            idx_v[b][s] = idx
            return 0

        lax.fori_loop(0, CHUNK // L, vec_body, 0, unroll=4)

    def press_and_write(ci, b):
        def press_body(g, _):
            t_r = [rows_v[b][g * L + t, :] for t in range(L)]
            for m in range(NCH):
                t0 = (L * m) // NCH
                acc = _rot(t_r[t0], L * m - NCH * t0)
                for d in (1, 2):
                    s_d = NCH * (t0 + d) - L * m
                    if s_d < L:
                        acc = jnp.where(
                            _iota() < s_d, acc,
                            _rot(t_r[t0 + d], L * m - NCH * (t0 + d)))
                pack_v[pl.ds(g * (L * NCH) + L * m, L)] = acc
            return 0

        lax.fori_loop(0, CHUNK // L, press_body, 0, unroll=2)
        cbase = base + ci * CHUNK
        pltpu.sync_copy(pack_v, out.at[pl.ds(cbase * NCH, CHUNK * NCH)])

    h_bi = start_bi(0, 0)
    h_gat = None
    for ci in range(NCHUNK):
        b = ci % 2
        for h in h_bi:
            h.wait()
        compute_idx(b)
        if ci + 1 < NCHUNK:
            h_bi = start_bi(ci + 1, 1 - b)
        h_next = pltpu.async_copy(table.at[idx_v[b]], rows_v[b], gat_sem[b])
        if h_gat is not None:
            press_and_write(ci - 1, 1 - b)
        h_gat = h_next
        h_gat.wait()
    press_and_write(NCHUNK - 1, (NCHUNK - 1) % 2)


def kernel(lod_cache, batch_index):
    regions = [
        lod_cache[k, : TEX_H >> k, : TEX_W >> k, :].reshape(-1)
        for k in range(7)
    ]
    rtail = jnp.concatenate(
        [lod_cache[k, : TEX_H >> k, : TEX_W >> k, :].reshape(-1)
         for k in range(7, NUM_LODS)] + [jnp.zeros((9,), jnp.float32)])
    bi = batch_index.astype(jnp.int32)
    flat = _tex_gather(*regions, rtail, bi[:, 0], bi[:, 1], bi[:, 2])
    return flat.reshape(BATCH, NCH)


# speedup vs baseline: 1.2033x; 1.2033x over previous
import functools

import jax
import jax.numpy as jnp
from jax import lax
from jax.experimental import pallas as pl
from jax.experimental.pallas import tpu as pltpu
from jax.experimental.pallas import tpu_sc as plsc

NUM_LODS = 10
TEX_H = 512
TEX_W = 512
NCH = 11
BATCH = 1048576

CPAD = 16
TSLAB = 1 << 19

NC = 2
NS = 16
L = 16
NW = NC * NS

BPW = BATCH // NW
CHUNK = 2048
NCHUNK = BPW // CHUNK

_mesh = plsc.VectorSubcoreMesh(core_axis_name="c", subcore_axis_name="s")


def _iota():
    return lax.iota(jnp.int32, L)


_GDN = lax.GatherDimensionNumbers(
    offset_dims=(), collapsed_slice_dims=(0,), start_index_map=(0,))


def _rot(v, sh):
    idx = (_iota() + (sh % L)) & (L - 1)
    return lax.gather(v, idx[:, None], _GDN, (1,),
                      mode=lax.GatherScatterMode.PROMISE_IN_BOUNDS)


@functools.partial(
    pl.kernel,
    out_type=jax.ShapeDtypeStruct((BATCH, CPAD), jnp.float32),
    mesh=_mesh,
    compiler_params=pltpu.CompilerParams(use_tc_tiling_on_sc=False),
    scratch_types=[
        pltpu.HBM((NC * TSLAB, CPAD), jnp.float32),
        [pltpu.VMEM((TEX_W * NCH,), jnp.float32) for _ in range(2)],
        pltpu.VMEM((TEX_W, CPAD), jnp.float32),
        [pltpu.VMEM((CHUNK * 3,), jnp.int32) for _ in range(2)],
        [pltpu.VMEM((CHUNK,), jnp.int32) for _ in range(2)],
        [pltpu.VMEM((CHUNK, CPAD), jnp.float32) for _ in range(2)],
        pltpu.VMEM((CHUNK * NCH,), jnp.float32),
        pltpu.SemaphoreType.DMA,
        [pltpu.SemaphoreType.DMA for _ in range(2)],
        [pltpu.SemaphoreType.DMA for _ in range(2)],
    ],
)
def _tex_gather(r0, r1, r2, r3, r4, r5, r6, rtail, ys, xs, lods, out,
                table, stage_in, stage_out, bi_v, idx_v, rows_v, pack_v,
                rep_sem, bi_sem, gat_sem):
    regions = (r0, r1, r2, r3, r4, r5, r6)
    cid = lax.axis_index("c")
    sid = lax.axis_index("s")
    wid = sid * NC + cid
    slab = cid * TSLAB

    def expand_group(g, ntex, sbuf, lane_off=0):
        a = [sbuf[pl.ds(lane_off + g * (L * NCH) + L * v, L)]
             for v in range(NCH)]
        for jj in range(min(L, ntex)):
            off = (NCH * jj) % L
            v0 = (NCH * jj) // L
            r = _rot(a[v0], off)
            if off > L - NCH:
                r = jnp.where(_iota() < (L - off), r, _rot(a[v0 + 1], off))
            stage_out[g * L + jj, :] = r

    def expand_and_store(y, wk, base_k, sbuf):
        if wk >= L:
            def grp(g, _):
                expand_group(g, L, sbuf)
                return 0
            lax.fori_loop(0, wk // L, grp, 0)
        else:
            expand_group(0, wk, sbuf)
        pltpu.sync_copy(
            stage_out.at[pl.ds(0, wk)],
            table.at[pl.ds(slab + base_k + y * wk, wk)],
        )

    for k in range(7):
        wk = TEX_W >> k
        base_k = TSLAB - (TSLAB >> k)
        reg = regions[k]
        n = wk * NCH
        if wk >= NS:
            rows_per = wk // NS
            y0 = sid * rows_per
            if rows_per >= 2:
                h = pltpu.async_copy(
                    reg.at[pl.ds(y0 * n, n)],
                    stage_in[0].at[pl.ds(0, n)], rep_sem)

                def rep2(r2i, _, wk=wk, base_k=base_k, reg=reg, n=n, y0=y0,
                         rows_per=rows_per):
                    y = y0 + r2i * 2
                    pltpu.make_async_copy(
                        reg.at[pl.ds(y * n, n)],
                        stage_in[0].at[pl.ds(0, n)], rep_sem).wait()
                    pltpu.async_copy(
                        reg.at[pl.ds((y + 1) * n, n)],
                        stage_in[1].at[pl.ds(0, n)], rep_sem)
                    expand_and_store(y, wk, base_k, stage_in[0])
                    pltpu.make_async_copy(
                        reg.at[pl.ds((y + 1) * n, n)],
                        stage_in[1].at[pl.ds(0, n)], rep_sem).wait()

                    @pl.when(r2i + 1 < rows_per // 2)
                    def _():
                        pltpu.async_copy(
                            reg.at[pl.ds((y + 2) * n, n)],
                            stage_in[0].at[pl.ds(0, n)], rep_sem)
                    expand_and_store(y + 1, wk, base_k, stage_in[1])
                    return 0

                lax.fori_loop(0, rows_per // 2, rep2, 0)
            else:
                pltpu.sync_copy(reg.at[pl.ds(y0 * n, n)],
                                stage_in[0].at[pl.ds(0, n)])
                expand_and_store(y0, wk, base_k, stage_in[0])
        else:
            @pl.when(sid < wk)
            def _(wk=wk, base_k=base_k, reg=reg, n=n):
                pltpu.sync_copy(reg.at[pl.ds(sid * n, n)],
                                stage_in[0].at[pl.ds(0, n)])
                expand_and_store(sid, wk, base_k, stage_in[0])

    @pl.when(sid == 0)
    def _():
        pltpu.sync_copy(rtail.at[pl.ds(0, 176)], stage_in[0].at[pl.ds(0, 176)])
        expand_group(0, 16, stage_in[0])
        pltpu.sync_copy(stage_out.at[pl.ds(0, 16)],
                        table.at[pl.ds(slab + TSLAB - (TSLAB >> 7), 16)])

    @pl.when(sid == 1)
    def _():
        pltpu.sync_copy(rtail.at[pl.ds(176, 48)], stage_in[0].at[pl.ds(0, 48)])
        expand_group(0, 4, stage_in[0])
        pltpu.sync_copy(stage_out.at[pl.ds(0, 4)],
                        table.at[pl.ds(slab + TSLAB - (TSLAB >> 8), 4)])

    @pl.when(sid == 2)
    def _():
        pltpu.sync_copy(rtail.at[pl.ds(216, 16)], stage_in[0].at[pl.ds(0, 16)])
        stage_out[0, :] = _rot(stage_in[0][pl.ds(0, L)], 4)
        pltpu.sync_copy(stage_out.at[pl.ds(0, 1)],
                        table.at[pl.ds(slab + TSLAB - (TSLAB >> 9), 1)])

    plsc.subcore_barrier()

    base = wid * BPW

    def start_bi(ci, b):
        cbase = base + ci * CHUNK
        return (
            pltpu.async_copy(ys.at[pl.ds(cbase, CHUNK)],
                             bi_v[b].at[pl.ds(0, CHUNK)], bi_sem[b]),
            pltpu.async_copy(xs.at[pl.ds(cbase, CHUNK)],
                             bi_v[b].at[pl.ds(CHUNK, CHUNK)], bi_sem[b]),
            pltpu.async_copy(lods.at[pl.ds(cbase, CHUNK)],
                             bi_v[b].at[pl.ds(2 * CHUNK, CHUNK)], bi_sem[b]),
        )

    def compute_idx(b):
        def vec_body(vi, _):
            s = pl.ds(vi * L, L)
            y = bi_v[b][s]
            x = bi_v[b][pl.ds(CHUNK + vi * L, L)]
            ld = bi_v[b][pl.ds(2 * CHUNK + vi * L, L)]
            lbase = TSLAB - lax.shift_right_logical(
                jnp.full((L,), TSLAB, jnp.int32), ld)
            w = lax.shift_right_logical(jnp.full((L,), TEX_W, jnp.int32), ld)
            idx = (
                slab + lbase
                + lax.shift_right_logical(y, ld) * w
                + lax.shift_right_logical(x, ld)
            )
            idx_v[b][s] = idx
            return 0

        lax.fori_loop(0, CHUNK // L, vec_body, 0, unroll=4)

    def press_and_write(ci, b):
        cbase = base + ci * CHUNK
        pltpu.sync_copy(rows_v[b], out.at[pl.ds(cbase, CHUNK)])

    h_bi = start_bi(0, 0)
    h_gat = None
    for ci in range(NCHUNK):
        b = ci % 2
        for h in h_bi:
            h.wait()
        compute_idx(b)
        if ci + 1 < NCHUNK:
            h_bi = start_bi(ci + 1, 1 - b)
        h_next = pltpu.async_copy(table.at[idx_v[b]], rows_v[b], gat_sem[b])
        if h_gat is not None:
            press_and_write(ci - 1, 1 - b)
        h_gat = h_next
        h_gat.wait()
    press_and_write(NCHUNK - 1, (NCHUNK - 1) % 2)


def kernel(lod_cache, batch_index):
    regions = [
        lod_cache[k, : TEX_H >> k, : TEX_W >> k, :].reshape(-1)
        for k in range(7)
    ]
    rtail = jnp.concatenate(
        [lod_cache[k, : TEX_H >> k, : TEX_W >> k, :].reshape(-1)
         for k in range(7, NUM_LODS)] + [jnp.zeros((9,), jnp.float32)])
    bi = batch_index.astype(jnp.int32)
    out16 = _tex_gather(*regions, rtail, bi[:, 0], bi[:, 1], bi[:, 2])
    return out16[:, :NCH]


# speedup vs baseline: 1.2206x; 1.0144x over previous
import functools

import jax
import jax.numpy as jnp
from jax import lax
from jax.experimental import pallas as pl
from jax.experimental.pallas import tpu as pltpu
from jax.experimental.pallas import tpu_sc as plsc

NUM_LODS = 10
TEX_H = 512
TEX_W = 512
NCH = 11
BATCH = 1048576

CPAD = 16
TSLAB = 1 << 19

NC = 2
NS = 16
L = 16
NW = NC * NS

BPW = BATCH // NW
CHUNK = 2048
NCHUNK = BPW // CHUNK

_STAGE = 11264

_mesh = plsc.VectorSubcoreMesh(core_axis_name="c", subcore_axis_name="s")


def _iota():
    return lax.iota(jnp.int32, L)


_GDN = lax.GatherDimensionNumbers(
    offset_dims=(), collapsed_slice_dims=(0,), start_index_map=(0,))


def _rot(v, sh):
    idx = (_iota() + (sh % L)) & (L - 1)
    return lax.gather(v, idx[:, None], _GDN, (1,),
                      mode=lax.GatherScatterMode.PROMISE_IN_BOUNDS)


@functools.partial(
    pl.kernel,
    out_type=jax.ShapeDtypeStruct((BATCH, CPAD), jnp.float32),
    mesh=_mesh,
    compiler_params=pltpu.CompilerParams(use_tc_tiling_on_sc=False),
    scratch_types=[
        pltpu.HBM((NC * TSLAB, CPAD), jnp.float32),
        [pltpu.VMEM((_STAGE,), jnp.float32) for _ in range(2)],
        pltpu.VMEM((_STAGE // NCH, CPAD), jnp.float32),
        [pltpu.VMEM((CHUNK * 3,), jnp.int32) for _ in range(2)],
        [pltpu.VMEM((CHUNK,), jnp.int32) for _ in range(2)],
        [pltpu.VMEM((CHUNK, CPAD), jnp.float32) for _ in range(2)],
        pltpu.SemaphoreType.DMA,
        [pltpu.SemaphoreType.DMA for _ in range(2)],
        [pltpu.SemaphoreType.DMA for _ in range(2)],
    ],
)
def _tex_gather(r0, r1, r2, r3, r4, r5, r6, rtail, ys, xs, lods, out,
                table, stage_in, stage_out, bi_v, idx_v, rows_v,
                rep_sem, bi_sem, gat_sem):
    regions = (r0, r1, r2, r3, r4, r5, r6)
    cid = lax.axis_index("c")
    sid = lax.axis_index("s")
    wid = sid * NC + cid
    slab = cid * TSLAB

    def expand_group(g, ntex, sbuf, lane_off=0):
        a = [sbuf[pl.ds(lane_off + g * (L * NCH) + L * v, L)]
             for v in range(NCH)]
        for jj in range(min(L, ntex)):
            off = (NCH * jj) % L
            v0 = (NCH * jj) // L
            r = _rot(a[v0], off)
            if off > L - NCH:
                r = jnp.where(_iota() < (L - off), r, _rot(a[v0 + 1], off))
            stage_out[g * L + jj, :] = r

    def expand_and_store(row0, nrows, wk, base_k, sbuf):
        ntex = nrows * wk
        if ntex >= L:
            def grp(g, _):
                expand_group(g, L, sbuf)
                return 0
            lax.fori_loop(0, ntex // L, grp, 0)
        else:
            expand_group(0, ntex, sbuf)
        pltpu.sync_copy(
            stage_out.at[pl.ds(0, ntex)],
            table.at[pl.ds(slab + base_k + row0 * wk, ntex)],
        )

    for k in range(7):
        wk = TEX_W >> k
        base_k = TSLAB - (TSLAB >> k)
        reg = regions[k]
        n = wk * NCH
        if wk >= NS:
            nr = wk // NS
            rj = min(nr, max(1, _STAGE // n))
            nj = nr // rj
            y0 = sid * nr
            ne = rj * n
            if nj >= 2:
                pltpu.async_copy(
                    reg.at[pl.ds(y0 * n, ne)],
                    stage_in[0].at[pl.ds(0, ne)], rep_sem)

                def rep2(j2, _, wk=wk, base_k=base_k, reg=reg, n=n, y0=y0,
                         rj=rj, nj=nj, ne=ne):
                    y = y0 + j2 * 2 * rj
                    pltpu.make_async_copy(
                        reg.at[pl.ds(y * n, ne)],
                        stage_in[0].at[pl.ds(0, ne)], rep_sem).wait()
                    pltpu.async_copy(
                        reg.at[pl.ds((y + rj) * n, ne)],
                        stage_in[1].at[pl.ds(0, ne)], rep_sem)
                    expand_and_store(y, rj, wk, base_k, stage_in[0])
                    pltpu.make_async_copy(
                        reg.at[pl.ds((y + rj) * n, ne)],
                        stage_in[1].at[pl.ds(0, ne)], rep_sem).wait()

                    @pl.when(j2 + 1 < nj // 2)
                    def _():
                        pltpu.async_copy(
                            reg.at[pl.ds((y + 2 * rj) * n, ne)],
                            stage_in[0].at[pl.ds(0, ne)], rep_sem)
                    expand_and_store(y + rj, rj, wk, base_k, stage_in[1])
                    return 0

                lax.fori_loop(0, nj // 2, rep2, 0)
            else:
                pltpu.sync_copy(reg.at[pl.ds(y0 * n, ne)],
                                stage_in[0].at[pl.ds(0, ne)])
                expand_and_store(y0, rj, wk, base_k, stage_in[0])
        else:
            @pl.when(sid < wk)
            def _(wk=wk, base_k=base_k, reg=reg, n=n):
                pltpu.sync_copy(reg.at[pl.ds(sid * n, n)],
                                stage_in[0].at[pl.ds(0, n)])
                expand_and_store(sid, 1, wk, base_k, stage_in[0])

    @pl.when(sid == 0)
    def _():
        pltpu.sync_copy(rtail.at[pl.ds(0, 176)], stage_in[0].at[pl.ds(0, 176)])
        expand_group(0, 16, stage_in[0])
        pltpu.sync_copy(stage_out.at[pl.ds(0, 16)],
                        table.at[pl.ds(slab + TSLAB - (TSLAB >> 7), 16)])

    @pl.when(sid == 1)
    def _():
        pltpu.sync_copy(rtail.at[pl.ds(176, 48)], stage_in[0].at[pl.ds(0, 48)])
        expand_group(0, 4, stage_in[0])
        pltpu.sync_copy(stage_out.at[pl.ds(0, 4)],
                        table.at[pl.ds(slab + TSLAB - (TSLAB >> 8), 4)])

    @pl.when(sid == 2)
    def _():
        pltpu.sync_copy(rtail.at[pl.ds(216, 16)], stage_in[0].at[pl.ds(0, 16)])
        stage_out[0, :] = _rot(stage_in[0][pl.ds(0, L)], 4)
        pltpu.sync_copy(stage_out.at[pl.ds(0, 1)],
                        table.at[pl.ds(slab + TSLAB - (TSLAB >> 9), 1)])

    plsc.subcore_barrier()

    base = wid * BPW

    def start_bi(ci, b):
        cbase = base + ci * CHUNK
        return (
            pltpu.async_copy(ys.at[pl.ds(cbase, CHUNK)],
                             bi_v[b].at[pl.ds(0, CHUNK)], bi_sem[b]),
            pltpu.async_copy(xs.at[pl.ds(cbase, CHUNK)],
                             bi_v[b].at[pl.ds(CHUNK, CHUNK)], bi_sem[b]),
            pltpu.async_copy(lods.at[pl.ds(cbase, CHUNK)],
                             bi_v[b].at[pl.ds(2 * CHUNK, CHUNK)], bi_sem[b]),
        )

    def compute_idx(b):
        def vec_body(vi, _):
            s = pl.ds(vi * L, L)
            y = bi_v[b][s]
            x = bi_v[b][pl.ds(CHUNK + vi * L, L)]
            ld = bi_v[b][pl.ds(2 * CHUNK + vi * L, L)]
            lbase = TSLAB - lax.shift_right_logical(
                jnp.full((L,), TSLAB, jnp.int32), ld)
            w = lax.shift_right_logical(jnp.full((L,), TEX_W, jnp.int32), ld)
            idx = (
                slab + lbase
                + lax.shift_right_logical(y, ld) * w
                + lax.shift_right_logical(x, ld)
            )
            idx_v[b][s] = idx
            return 0

        lax.fori_loop(0, CHUNK // L, vec_body, 0, unroll=4)

    def press_and_write(ci, b):
        cbase = base + ci * CHUNK
        pltpu.sync_copy(rows_v[b], out.at[pl.ds(cbase, CHUNK)])

    h_bi = start_bi(0, 0)
    h_gat = None
    for ci in range(NCHUNK):
        b = ci % 2
        for h in h_bi:
            h.wait()
        compute_idx(b)
        if ci + 1 < NCHUNK:
            h_bi = start_bi(ci + 1, 1 - b)
        h_next = pltpu.async_copy(table.at[idx_v[b]], rows_v[b], gat_sem[b])
        if h_gat is not None:
            press_and_write(ci - 1, 1 - b)
        h_gat = h_next
        h_gat.wait()
    press_and_write(NCHUNK - 1, (NCHUNK - 1) % 2)


def kernel(lod_cache, batch_index):
    regions = [
        lod_cache[k, : TEX_H >> k, : TEX_W >> k, :].reshape(-1)
        for k in range(7)
    ]
    rtail = jnp.concatenate(
        [lod_cache[k, : TEX_H >> k, : TEX_W >> k, :].reshape(-1)
         for k in range(7, NUM_LODS)] + [jnp.zeros((9,), jnp.float32)])
    bi = batch_index.astype(jnp.int32)
    out16 = _tex_gather(*regions, rtail, bi[:, 0], bi[:, 1], bi[:, 2])
    return out16[:, :NCH]
